# Initial kernel scaffold; baseline (speedup 1.0000x reference)
import jax
import jax.numpy as jnp
from jax.experimental import pallas as pl

B, D, M, K = 4096, 256, 8, 1024
BB = 256


def _drq_kernel(x_ref, cb_ref, recon_ref, codes_ref, side_ref):
    x = x_ref[...]
    x_recon = jnp.zeros_like(x)
    iota = jax.lax.broadcasted_iota(jnp.int32, (BB, K), 1)
    for m in range(M):
        cb = cb_ref[m]
        cbnorm = jnp.sum(cb * cb, axis=1)
        res = x - x_recon
        scores = jax.lax.dot_general(
            res, cb, (((1,), (1,)), ((), ())),
            preferred_element_type=jnp.float32)
        dist = cbnorm[None, :] - 2.0 * scores
        idx = jnp.argmin(dist, axis=1)
        onehot = (iota == idx[:, None]).astype(jnp.float32)
        recon = jax.lax.dot_general(
            onehot, cb, (((1,), (0,)), ((), ())),
            preferred_element_type=jnp.float32)
        x_recon = x_recon + recon
        codes_ref[:, m * K:(m + 1) * K] = onehot
        side_ref[m, :, :] = x_recon
    recon_ref[...] = x_recon


@jax.jit
def kernel(x, codebooks):
    cb = codebooks.reshape(M, K, D)
    recon, codes_flat, side = pl.pallas_call(
        _drq_kernel,
        grid=(B // BB,),
        in_specs=[
            pl.BlockSpec((BB, D), lambda i: (i, 0)),
            pl.BlockSpec((M, K, D), lambda i: (0, 0, 0)),
        ],
        out_specs=[
            pl.BlockSpec((BB, D), lambda i: (i, 0)),
            pl.BlockSpec((BB, M * K), lambda i: (i, 0)),
            pl.BlockSpec((M, BB, D), lambda i: (0, i, 0)),
        ],
        out_shape=[
            jax.ShapeDtypeStruct((B, D), jnp.float32),
            jax.ShapeDtypeStruct((B, M * K), jnp.float32),
            jax.ShapeDtypeStruct((M, B, D), jnp.float32),
        ],
    )(x, cb)
    return recon, codes_flat.reshape(B, M, K), side


# speedup vs baseline: 2.8388x; 2.8388x over previous
import jax
import jax.numpy as jnp
from jax.experimental import pallas as pl
from jax.experimental.pallas import tpu as pltpu

B, D, M, K = 4096, 256, 8, 1024
BB = 256


def _drq_kernel(x_ref, cb_ref, recon_ref, codes_ref, side_ref, xrec_ref):
    m = pl.program_id(0)
    i = pl.program_id(1)
    rows = pl.ds(i * BB, BB)
    xb = x_ref[rows, :]
    prev = jnp.where(m == 0, 0.0, xrec_ref[rows, :])
    res = xb - prev

    cb = cb_ref[0]
    sq = cb * cb
    cbnorm = jax.lax.dot_general(
        jnp.ones((1, D), jnp.float32), sq, (((1,), (1,)), ((), ())),
        preferred_element_type=jnp.float32)
    scores = jax.lax.dot_general(
        res, cb, (((1,), (1,)), ((), ())),
        preferred_element_type=jnp.float32)
    dist = cbnorm - 2.0 * scores
    idx = jnp.argmin(dist, axis=1)
    iota = jax.lax.broadcasted_iota(jnp.int32, (BB, K), 1)
    onehot = (iota == idx[:, None]).astype(jnp.float32)
    recon = jax.lax.dot_general(
        onehot, cb, (((1,), (0,)), ((), ())),
        preferred_element_type=jnp.float32)
    new = prev + recon

    xrec_ref[rows, :] = new
    codes_ref[...] = onehot
    side_ref[0, :, :] = new
    recon_ref[...] = new


@jax.jit
def kernel(x, codebooks):
    cb = codebooks.reshape(M, K, D)
    recon, codes_flat, side = pl.pallas_call(
        _drq_kernel,
        grid=(M, B // BB),
        in_specs=[
            pl.BlockSpec((B, D), lambda m, i: (0, 0)),
            pl.BlockSpec((1, K, D), lambda m, i: (m, 0, 0)),
        ],
        out_specs=[
            pl.BlockSpec((BB, D), lambda m, i: (i, 0)),
            pl.BlockSpec((BB, K), lambda m, i: (i, m)),
            pl.BlockSpec((1, BB, D), lambda m, i: (m, i, 0)),
        ],
        out_shape=[
            jax.ShapeDtypeStruct((B, D), jnp.float32),
            jax.ShapeDtypeStruct((B, M * K), jnp.float32),
            jax.ShapeDtypeStruct((M, B, D), jnp.float32),
        ],
        scratch_shapes=[pltpu.VMEM((B, D), jnp.float32)],
    )(x, cb)
    return recon, codes_flat.reshape(B, M, K), side


# speedup vs baseline: 3.1504x; 1.1098x over previous
import jax
import jax.numpy as jnp
from jax.experimental import pallas as pl
from jax.experimental.pallas import tpu as pltpu

B, D, M, K = 4096, 256, 8, 1024
BB = 512


def _drq_kernel(x_ref, cb_ref, recon_ref, codes_ref, side_ref,
                xrec_ref, cbn_ref):
    m = pl.program_id(0)
    i = pl.program_id(1)
    rows = pl.ds(i * BB, BB)
    cb = cb_ref[0]

    @pl.when(i == 0)
    def _():
        sq = cb * cb
        ones = jnp.ones((8, D), jnp.float32)

        def _rowdot(a):
            return jax.lax.dot_general(
                ones, a, (((1,), (1,)), ((), ())),
                preferred_element_type=jnp.float32)

        hi = sq.astype(jnp.bfloat16).astype(jnp.float32)
        r1 = sq - hi
        mid = r1.astype(jnp.bfloat16).astype(jnp.float32)
        lo = r1 - mid
        cbn_ref[...] = (_rowdot(hi) + _rowdot(mid)) + _rowdot(lo)

    xb = x_ref[rows, :]
    prev = jnp.where(m == 0, 0.0, xrec_ref[rows, :])
    res = xb - prev
    xx = jnp.sum(res * res, axis=1, keepdims=True)
    scores = jax.lax.dot_general(
        res, cb, (((1,), (1,)), ((), ())),
        preferred_element_type=jnp.float32)
    dist = (xx - 2.0 * scores) + cbn_ref[0:1, :]
    iota = jax.lax.broadcasted_iota(jnp.int32, (BB, K), 1)
    rowmin = jnp.min(dist, axis=1, keepdims=True)
    idx = jnp.min(jnp.where(dist == rowmin, iota, K), axis=1, keepdims=True)
    onehot = (iota == idx).astype(jnp.float32)
    recon = jax.lax.dot_general(
        onehot, cb, (((1,), (0,)), ((), ())),
        preferred_element_type=jnp.float32)
    new = prev + recon

    xrec_ref[rows, :] = new
    codes_ref[...] = onehot
    side_ref[0, :, :] = new
    recon_ref[...] = new


@jax.jit
def kernel(x, codebooks):
    cb = codebooks.reshape(M, K, D)
    recon, codes_flat, side = pl.pallas_call(
        _drq_kernel,
        grid=(M, B // BB),
        in_specs=[
            pl.BlockSpec((B, D), lambda m, i: (0, 0)),
            pl.BlockSpec((1, K, D), lambda m, i: (m, 0, 0)),
        ],
        out_specs=[
            pl.BlockSpec((BB, D), lambda m, i: (i, 0)),
            pl.BlockSpec((BB, K), lambda m, i: (i, m)),
            pl.BlockSpec((1, BB, D), lambda m, i: (m, i, 0)),
        ],
        out_shape=[
            jax.ShapeDtypeStruct((B, D), jnp.float32),
            jax.ShapeDtypeStruct((B, M * K), jnp.float32),
            jax.ShapeDtypeStruct((M, B, D), jnp.float32),
        ],
        scratch_shapes=[
            pltpu.VMEM((B, D), jnp.float32),
            pltpu.VMEM((8, K), jnp.float32),
        ],
    )(x, cb)
    return recon, codes_flat.reshape(B, M, K), side


# speedup vs baseline: 3.5576x; 1.1293x over previous
import jax
import jax.numpy as jnp
from jax.experimental import pallas as pl
from jax.experimental.pallas import tpu as pltpu

B, D, M, K = 4096, 256, 8, 1024
BB = 2048


def _drq_kernel(x_ref, cb_ref, recon_ref, codes_ref, side_ref,
                xrec_ref, cbn_ref):
    m = pl.program_id(0)
    i = pl.program_id(1)
    rows = pl.ds(i * BB, BB)
    cb = cb_ref[0]

    @pl.when(i == 0)
    def _():
        sq = cb * cb
        ones = jnp.ones((8, D), jnp.float32)

        def _rowdot(a):
            return jax.lax.dot_general(
                ones, a, (((1,), (1,)), ((), ())),
                preferred_element_type=jnp.float32)

        hi = sq.astype(jnp.bfloat16).astype(jnp.float32)
        r1 = sq - hi
        mid = r1.astype(jnp.bfloat16).astype(jnp.float32)
        lo = r1 - mid
        cbn_ref[...] = (_rowdot(hi) + _rowdot(mid)) + _rowdot(lo)

    xb = x_ref[rows, :]
    prev = jnp.where(m == 0, 0.0, xrec_ref[rows, :])
    res = xb - prev
    xx = jnp.sum(res * res, axis=1, keepdims=True)
    scores = jax.lax.dot_general(
        res, cb, (((1,), (1,)), ((), ())),
        preferred_element_type=jnp.float32)
    dist = (xx - 2.0 * scores) + cbn_ref[0:1, :]
    iota = jax.lax.broadcasted_iota(jnp.int32, (BB, K), 1)
    rowmin = jnp.min(dist, axis=1, keepdims=True)
    idx = jnp.min(jnp.where(dist == rowmin, iota, K), axis=1, keepdims=True)
    onehot = (iota == idx).astype(jnp.float32)
    recon = jax.lax.dot_general(
        onehot, cb, (((1,), (0,)), ((), ())),
        preferred_element_type=jnp.float32)
    new = prev + recon

    xrec_ref[rows, :] = new
    codes_ref[...] = onehot
    side_ref[0, :, :] = new
    recon_ref[...] = new


@jax.jit
def kernel(x, codebooks):
    cb = codebooks.reshape(M, K, D)
    recon, codes_flat, side = pl.pallas_call(
        _drq_kernel,
        grid=(M, B // BB),
        in_specs=[
            pl.BlockSpec((B, D), lambda m, i: (0, 0)),
            pl.BlockSpec((1, K, D), lambda m, i: (m, 0, 0)),
        ],
        out_specs=[
            pl.BlockSpec((BB, D), lambda m, i: (i, 0)),
            pl.BlockSpec((BB, K), lambda m, i: (i, m)),
            pl.BlockSpec((1, BB, D), lambda m, i: (m, i, 0)),
        ],
        out_shape=[
            jax.ShapeDtypeStruct((B, D), jnp.float32),
            jax.ShapeDtypeStruct((B, M * K), jnp.float32),
            jax.ShapeDtypeStruct((M, B, D), jnp.float32),
        ],
        scratch_shapes=[
            pltpu.VMEM((B, D), jnp.float32),
            pltpu.VMEM((8, K), jnp.float32),
        ],
    )(x, cb)
    return recon, codes_flat.reshape(B, M, K), side


# speedup vs baseline: 3.9418x; 1.1080x over previous
import jax
import jax.numpy as jnp
from jax.experimental import pallas as pl
from jax.experimental.pallas import tpu as pltpu

B, D, M, K = 4096, 256, 8, 1024
BB = 512


def _drq_kernel(x_ref, cb_ref, recon_ref, codes_ref, side_ref,
                xrec_ref, cbn_ref):
    i = pl.program_id(0)
    m = pl.program_id(1)
    rows = pl.ds(i * BB, BB)
    cb = cb_ref[m]

    @pl.when(i == 0)
    def _():
        sq = cb * cb
        ones = jnp.ones((8, D), jnp.float32)

        def _rowdot(a):
            return jax.lax.dot_general(
                ones, a, (((1,), (1,)), ((), ())),
                preferred_element_type=jnp.float32)

        hi = sq.astype(jnp.bfloat16).astype(jnp.float32)
        r1 = sq - hi
        mid = r1.astype(jnp.bfloat16).astype(jnp.float32)
        lo = r1 - mid
        cbn_ref[m] = (_rowdot(hi) + _rowdot(mid)) + _rowdot(lo)

    xb = x_ref[rows, :]
    prev = jnp.where(m == 0, 0.0, xrec_ref[...])
    res = xb - prev
    xx = jnp.sum(res * res, axis=1, keepdims=True)
    scores = jax.lax.dot_general(
        res, cb, (((1,), (1,)), ((), ())),
        preferred_element_type=jnp.float32)
    dist = (xx - 2.0 * scores) + cbn_ref[m, 0:1, :]
    iota = jax.lax.broadcasted_iota(jnp.int32, (BB, K), 1)
    rowmin = jnp.min(dist, axis=1, keepdims=True)
    idx = jnp.min(jnp.where(dist == rowmin, iota, K), axis=1, keepdims=True)
    onehot = (iota == idx).astype(jnp.float32)
    recon = jax.lax.dot_general(
        onehot, cb, (((1,), (0,)), ((), ())),
        preferred_element_type=jnp.float32)
    new = prev + recon

    xrec_ref[...] = new
    codes_ref[:, pl.ds(m, 1), :] = onehot[:, None, :]
    side_ref[0, :, :] = new
    recon_ref[...] = new


@jax.jit
def kernel(x, codebooks):
    cb = codebooks.reshape(M, K, D)
    recon, codes, side = pl.pallas_call(
        _drq_kernel,
        grid=(B // BB, M),
        in_specs=[
            pl.BlockSpec((B, D), lambda i, m: (0, 0)),
            pl.BlockSpec((M, K, D), lambda i, m: (0, 0, 0)),
        ],
        out_specs=[
            pl.BlockSpec((BB, D), lambda i, m: (i, 0)),
            pl.BlockSpec((BB, M, K), lambda i, m: (i, 0, 0)),
            pl.BlockSpec((1, BB, D), lambda i, m: (m, i, 0)),
        ],
        out_shape=[
            jax.ShapeDtypeStruct((B, D), jnp.float32),
            jax.ShapeDtypeStruct((B, M, K), jnp.float32),
            jax.ShapeDtypeStruct((M, B, D), jnp.float32),
        ],
        scratch_shapes=[
            pltpu.VMEM((BB, D), jnp.float32),
            pltpu.VMEM((M, 8, K), jnp.float32),
        ],
    )(x, cb)
    return recon, codes, side
